# baseline (speedup 1.0000x reference)
import functools

import jax
import jax.numpy as jnp
from jax import lax
from jax.experimental import pallas as pl
from jax.experimental.pallas import tpu as pltpu
from jax.experimental.pallas import tpu_sc as plsc

B, F, V, D = 16384, 26, 1000000, 32
NC, NS = 2, 16
NW = NC * NS
RPW = B // NW
CH = 64
NCH = RPW // CH
CHF = CH * F
L = 16
NBUF = 2


def _fm_body(idx_hbm, table_hbm, w_hbm, out_hbm, idxbuf, ebuf, w_v, out_v,
             sems):
    cid = lax.axis_index("c")
    sid = lax.axis_index("s")
    wid = sid * NC + cid
    base = wid * RPW

    pltpu.sync_copy(w_hbm, w_v)
    lane = lax.iota(jnp.int32, L)
    w_lo = w_v[pl.ds(0, L)]
    w_hi = w_v[pl.ds(L, L)]
    b_scal = w_v[pl.ds(2 * L, L)][0]

    def fire(ch, k):
        off = pl.multiple_of((wid * NCH + ch) * CHF, CHF)
        pltpu.sync_copy(idx_hbm.at[pl.ds(off, CHF)],
                        idxbuf.at[pl.ds(k * CHF, CHF)])
        pltpu.async_copy(table_hbm.at[idxbuf.at[pl.ds(k * CHF, CHF)]],
                         ebuf.at[pl.ds(k * CHF, CHF)], sems.at[k])

    def wait(k):
        pltpu.make_async_copy(table_hbm.at[idxbuf.at[pl.ds(k * CHF, CHF)]],
                              ebuf.at[pl.ds(k * CHF, CHF)], sems.at[k]).wait()

    fire(0, 0)

    def chunk_body(ch, _):
        k = lax.rem(ch, NBUF)
        wait(k)

        @pl.when(ch + 1 < NCH)
        def _():
            fire(ch + 1, 1 - k)

        ebase = k * CHF

        def group_body(g, resvec):
            row0 = g * L

            def one_row(j):
                roff = (row0 + j) * F
                ulo = idxbuf[pl.ds(ebase + roff, L)]
                uhi = idxbuf[pl.ds(ebase + roff + L, L)]
                ivlo = ((ulo & ~2047) + ((ulo & 3) << 9)
                        + ((ulo >> 2) & 511)).astype(jnp.float32)
                ivhi = ((uhi & ~2047) + ((uhi & 3) << 9)
                        + ((uhi >> 2) & 511)).astype(jnp.float32)
                acc = ivlo * w_lo + ivhi * w_hi

                s0 = [None, None]
                s1 = [None, None]
                q0 = [None, None]
                q1 = [None, None]
                for f in range(F):
                    p = f & 1
                    e0 = ebuf[ebase + roff + f, pl.ds(0, L)]
                    e1 = ebuf[ebase + roff + f, pl.ds(L, L)]
                    if s0[p] is None:
                        s0[p], s1[p] = e0, e1
                        q0[p], q1[p] = e0 * e0, e1 * e1
                    else:
                        s0[p] = s0[p] + e0
                        s1[p] = s1[p] + e1
                        q0[p] = q0[p] + e0 * e0
                        q1[p] = q1[p] + e1 * e1
                ss0 = s0[0] + s0[1]
                ss1 = s1[0] + s1[1]
                fm = (ss0 * ss0 + ss1 * ss1) - (q0[0] + q0[1]) - (q1[0] + q1[1])
                return jnp.sum(acc + 0.5 * fm) + b_scal

            for j in range(L):
                resvec = jnp.where(lane == j, one_row(j), resvec)
            out_v[pl.ds(ch * CH + row0, L)] = resvec
            return resvec

        lax.fori_loop(0, CH // L, group_body, jnp.zeros((L,), jnp.float32))
        return 0

    lax.fori_loop(0, NCH, chunk_body, 0)
    pltpu.sync_copy(out_v, out_hbm.at[pl.ds(base, RPW)])


@functools.partial(jax.jit, static_argnames=())
def _fm_sc(idx_flat, table, wfull):
    run = pl.kernel(
        _fm_body,
        out_type=jax.ShapeDtypeStruct((B,), jnp.float32),
        name="fm_sc",
        mesh=plsc.VectorSubcoreMesh(
            core_axis_name="c", subcore_axis_name="s",
            num_cores=NC, num_subcores=NS),
        scratch_types=[
            pltpu.VMEM((NBUF * CHF + 8,), jnp.int32),
            pltpu.VMEM((NBUF * CHF, D), jnp.float32),
            pltpu.VMEM((3 * L,), jnp.float32),
            pltpu.VMEM((RPW,), jnp.float32),
            pltpu.SemaphoreType.DMA((NBUF,)),
        ],
        compiler_params=pltpu.CompilerParams(
            needs_layout_passes=False, use_tc_tiling_on_sc=False),
    )
    return run(idx_flat, table, wfull)


TRV = 512
VB = 4 * TRV
GT = (V + VB - 1) // VB
VP = GT * VB
NCB = (V + TRV - 1) // TRV - 1


def _tr_body(t0, t1, t2, t3, tout_ref):
    ident = (lax.broadcasted_iota(jnp.int32, (D, D), 0)
             == lax.broadcasted_iota(jnp.int32, (D, D), 1)).astype(jnp.float32)
    for k, tk in enumerate((t0, t1, t2, t3)):
        tout_ref[:, pl.ds(k * D, D)] = lax.dot_general(
            tk[...], ident, (((0,), (0,)), ((), ())),
            precision=lax.Precision.HIGHEST)


def _transpose_table(table):
    tr = pl.pallas_call(
        _tr_body,
        grid=(GT,),
        in_specs=[
            pl.BlockSpec((D, TRV),
                         lambda i, k=k: (0, jnp.minimum(4 * i + k, NCB)))
            for k in range(4)
        ],
        out_specs=pl.BlockSpec((TRV, 4 * D), lambda i: (i, 0)),
        out_shape=jax.ShapeDtypeStruct((VP // 4, 4 * D), jnp.float32),
    )
    tt = table.T
    return tr(tt, tt, tt, tt).reshape(VP, D)


def kernel(interaction_pairs, table, W, b):
    idx = interaction_pairs.astype(jnp.int32)
    u = ((idx & ~2047) + ((idx & 511) << 2) + ((idx >> 9) & 3))
    idx_flat = u.reshape(B * F)
    wfull = jnp.concatenate(
        [W[:, 0].astype(jnp.float32),
         jnp.zeros((2 * L - F,), jnp.float32),
         jnp.broadcast_to(b.astype(jnp.float32), (L,))])
    return _fm_sc(idx_flat, _transpose_table(table), wfull)


# speedup vs baseline: 4.2462x; 4.2462x over previous
import functools

import jax
import jax.numpy as jnp
from jax import lax
from jax.experimental import pallas as pl
from jax.experimental.pallas import tpu as pltpu
from jax.experimental.pallas import tpu_sc as plsc

B, F, V, D = 16384, 26, 1000000, 32
NC, NS = 2, 16
NW = NC * NS
RPW = B // NW
CH = 64
NCH = RPW // CH
CHF = CH * F
L = 16
NBUF = 2

TRV = 4096
SH = TRV.bit_length() - 1
VB = 4 * TRV
GT = (V + VB - 1) // VB
VP = GT * VB
NCB = (V + TRV - 1) // TRV - 1


def _perm(v):
    return (v & ~(VB - 1)) + ((v & (TRV - 1)) << 2) + ((v >> SH) & 3)


def _perm_inv(u):
    return (u & ~(VB - 1)) + ((u & 3) << SH) + ((u >> 2) & (TRV - 1))


def _fm_body(idx_hbm, table_hbm, w_hbm, out_hbm, idxbuf, ebuf, w_v, out_v,
             sems):
    cid = lax.axis_index("c")
    sid = lax.axis_index("s")
    wid = sid * NC + cid
    base = wid * RPW

    pltpu.sync_copy(w_hbm, w_v)
    lane = lax.iota(jnp.int32, L)
    w_lo = w_v[pl.ds(0, L)]
    w_hi = w_v[pl.ds(L, L)]
    b_scal = w_v[pl.ds(2 * L, L)][0]

    def fire(ch, k):
        off = pl.multiple_of((wid * NCH + ch) * CHF, CHF)
        pltpu.sync_copy(idx_hbm.at[pl.ds(off, CHF)],
                        idxbuf.at[pl.ds(k * CHF, CHF)])
        pltpu.async_copy(table_hbm.at[idxbuf.at[pl.ds(k * CHF, CHF)]],
                         ebuf.at[pl.ds(k * CHF, CHF)], sems.at[k])

    def wait(k):
        pltpu.make_async_copy(table_hbm.at[idxbuf.at[pl.ds(k * CHF, CHF)]],
                              ebuf.at[pl.ds(k * CHF, CHF)], sems.at[k]).wait()

    fire(0, 0)

    def chunk_body(ch, _):
        k = lax.rem(ch, NBUF)
        wait(k)

        @pl.when(ch + 1 < NCH)
        def _():
            fire(ch + 1, 1 - k)

        ebase = k * CHF

        def group_body(g, resvec):
            row0 = g * L

            def one_row(j):
                roff = (row0 + j) * F
                ulo = idxbuf[pl.ds(ebase + roff, L)]
                uhi = idxbuf[pl.ds(ebase + roff + L, L)]
                ivlo = _perm_inv(ulo).astype(jnp.float32)
                ivhi = _perm_inv(uhi).astype(jnp.float32)
                acc = ivlo * w_lo + ivhi * w_hi

                s0 = [None, None]
                s1 = [None, None]
                q0 = [None, None]
                q1 = [None, None]
                for f in range(F):
                    p = f & 1
                    e0 = ebuf[ebase + roff + f, pl.ds(0, L)]
                    e1 = ebuf[ebase + roff + f, pl.ds(L, L)]
                    if s0[p] is None:
                        s0[p], s1[p] = e0, e1
                        q0[p], q1[p] = e0 * e0, e1 * e1
                    else:
                        s0[p] = s0[p] + e0
                        s1[p] = s1[p] + e1
                        q0[p] = q0[p] + e0 * e0
                        q1[p] = q1[p] + e1 * e1
                ss0 = s0[0] + s0[1]
                ss1 = s1[0] + s1[1]
                fm = (ss0 * ss0 + ss1 * ss1) - (q0[0] + q0[1]) - (q1[0] + q1[1])
                return jnp.sum(acc + 0.5 * fm) + b_scal

            for j in range(L):
                resvec = jnp.where(lane == j, one_row(j), resvec)
            out_v[pl.ds(ch * CH + row0, L)] = resvec
            return resvec

        lax.fori_loop(0, CH // L, group_body, jnp.zeros((L,), jnp.float32))
        return 0

    lax.fori_loop(0, NCH, chunk_body, 0)
    pltpu.sync_copy(out_v, out_hbm.at[pl.ds(base, RPW)])


@functools.partial(jax.jit, static_argnames=())
def _fm_sc(idx_flat, table, wfull):
    run = pl.kernel(
        _fm_body,
        out_type=jax.ShapeDtypeStruct((B,), jnp.float32),
        name="fm_sc",
        mesh=plsc.VectorSubcoreMesh(
            core_axis_name="c", subcore_axis_name="s",
            num_cores=NC, num_subcores=NS),
        scratch_types=[
            pltpu.VMEM((NBUF * CHF + 8,), jnp.int32),
            pltpu.VMEM((NBUF * CHF, D), jnp.float32),
            pltpu.VMEM((3 * L,), jnp.float32),
            pltpu.VMEM((RPW,), jnp.float32),
            pltpu.SemaphoreType.DMA((NBUF,)),
        ],
        compiler_params=pltpu.CompilerParams(
            needs_layout_passes=False, use_tc_tiling_on_sc=False),
    )
    return run(idx_flat, table, wfull)


def _tr_body(t0, t1, t2, t3, tout_ref):
    x = jnp.concatenate([t0[...], t1[...], t2[...], t3[...]], axis=0)
    tout_ref[...] = jnp.transpose(x)


def _transpose_table(table):
    tr = pl.pallas_call(
        _tr_body,
        grid=(GT,),
        in_specs=[
            pl.BlockSpec((D, TRV),
                         lambda i, k=k: (0, jnp.minimum(4 * i + k, NCB)))
            for k in range(4)
        ],
        out_specs=pl.BlockSpec((TRV, 4 * D), lambda i: (i, 0)),
        out_shape=jax.ShapeDtypeStruct((VP // 4, 4 * D), jnp.float32),
    )
    tt = table.T
    return tr(tt, tt, tt, tt).reshape(VP, D)


def kernel(interaction_pairs, table, W, b):
    idx = interaction_pairs.astype(jnp.int32)
    idx_flat = _perm(idx).reshape(B * F)
    wfull = jnp.concatenate(
        [W[:, 0].astype(jnp.float32),
         jnp.zeros((2 * L - F,), jnp.float32),
         jnp.broadcast_to(b.astype(jnp.float32), (L,))])
    return _fm_sc(idx_flat, _transpose_table(table), wfull)
